# baseline (speedup 1.0000x reference)
import jax
import jax.numpy as jnp
from jax import lax
from jax.experimental import pallas as pl
from jax.experimental.pallas import tpu as pltpu
from jax.experimental.pallas import tpu_sc as plsc
from jax._src.pallas import mpmd as _plmpmd


def _bulk_body(x_ref, tgt_ref, sos_ref, xo_ref, to_ref):
    S = x_ref.shape[1]
    xo_ref[0, pl.ds(0, 1)] = sos_ref[0]
    xo_ref[0, pl.ds(1, S)] = x_ref[0]
    to_ref[0] = tgt_ref[0]


def _win_body(B, S, J, T_L,
              xp, tp, lens, c, sos, labels3, xo, to,
              win, lens_s, c_s, gidx, idx64, idx16, wsem):
    del xp, tp
    L = 16
    wid = lax.axis_index("s") * 2 + lax.axis_index("c")
    b = wid % B
    kind = wid // B

    pltpu.sync_copy(lens, lens_s)
    pltpu.sync_copy(c, c_s)
    lane = lax.iota(jnp.int32, L)
    bvec = jnp.full((L,), 0, jnp.int32) + b
    lbv = plsc.load_gather(lens_s, [bvec])
    cb = jnp.max(jnp.where(lane == b, c_s[...], 0), axis=0)

    gidx[...] = bvec
    gl = pltpu.async_copy(labels3.at[cb], win.at[pl.ds(0, T_L)], wsem)
    gs = pltpu.async_copy(sos.at[gidx], win.at[pl.ds(T_L, L)], wsem)
    gl.wait()
    gs.wait()

    @pl.when(kind == 0)
    def _():
        base = b * (S + 1)
        for k in range(T_L // L):
            idx64[pl.ds(k * L, L)] = lbv + (base + 2 + k * L) + lane
        idx16[...] = jnp.where(lane == 0, lbv + base + 1,
                               jnp.full((L,), 0, jnp.int32) + base)
        s1 = pltpu.async_copy(win.at[pl.ds(0, T_L)], xo.at[idx64], wsem)
        s2 = pltpu.async_copy(win.at[pl.ds(T_L, L)], xo.at[idx16], wsem)
        s1.wait()
        s2.wait()

    @pl.when(kind == 1)
    def _():
        base = b * S
        for k in range(T_L // L):
            idx64[pl.ds(k * L, L)] = lbv + (base + 1 + k * L) + lane
        idx16[...] = jnp.where(lane == 0, lbv + base, lbv + base + T_L + 1)
        s1 = pltpu.async_copy(win.at[pl.ds(0, T_L)], to.at[idx64], wsem)
        s2 = pltpu.async_copy(win.at[pl.ds(T_L, L)], to.at[idx16], wsem)
        s1.wait()
        s2.wait()


def kernel(x, tgt, lens, c, sos, labels):
    B, S, J = x.shape
    C, T_L = labels.shape[0], labels.shape[1]
    x_pre, t_pre = pl.pallas_call(
        _bulk_body,
        grid=(B,),
        out_shape=(
            jax.ShapeDtypeStruct((B, S + 1, J), x.dtype),
            jax.ShapeDtypeStruct((B, S, J), tgt.dtype),
        ),
        in_specs=[
            pl.BlockSpec((1, S, J), lambda b: (b, 0, 0)),
            pl.BlockSpec((1, S, J), lambda b: (b, 0, 0)),
            pl.BlockSpec((1, 1, J), lambda b: (b, 0, 0)),
        ],
        out_specs=(
            pl.BlockSpec((1, S + 1, J), lambda b: (b, 0, 0)),
            pl.BlockSpec((1, S, J), lambda b: (b, 0, 0)),
        ),
    )(x, tgt, sos.reshape(B, 1, J))

    win_call = _plmpmd._mpmd_map(
        [(plsc.VectorSubcoreMesh(core_axis_name="c", subcore_axis_name="s"),
          lambda *refs: _win_body(B, S, J, T_L, *refs))],
        (
            jax.ShapeDtypeStruct((B * (S + 1), J), x.dtype),
            jax.ShapeDtypeStruct((B * S, J), tgt.dtype),
        ),
        input_output_aliases={0: 0, 1: 1},
        scratch_types=[
            pltpu.VMEM((T_L + 16, J), x.dtype),
            pltpu.VMEM((B,), jnp.int32),
            pltpu.VMEM((B,), jnp.int32),
            pltpu.VMEM((16,), jnp.int32),
            pltpu.VMEM((T_L,), jnp.int32),
            pltpu.VMEM((16,), jnp.int32),
            pltpu.SemaphoreType.DMA,
        ],
        compiler_params=pltpu.CompilerParams(needs_layout_passes=False),
    )
    x_out, tgt_out = win_call(
        x_pre.reshape(B * (S + 1), J), t_pre.reshape(B * S, J),
        lens, c, sos, labels)
    return (x_out.reshape(B, S + 1, J), tgt_out.reshape(B, S, J), labels)


# speedup vs baseline: 1.0093x; 1.0093x over previous
import jax
import jax.numpy as jnp
from jax import lax
from jax.experimental import pallas as pl
from jax.experimental.pallas import tpu as pltpu
from jax.experimental.pallas import tpu_sc as plsc
from jax._src.pallas import mpmd as _plmpmd


def _bulk_body(x_ref, tgt_ref, sos_ref, xo_ref, to_ref):
    S = x_ref.shape[1]
    xo_ref[0, pl.ds(0, 1)] = sos_ref[0]
    xo_ref[0, pl.ds(1, S)] = x_ref[0]
    to_ref[0] = tgt_ref[0]


def _lcopy_body(C, T_L, J, labels3, lout, buf, *sems):
    D = 3
    wid = lax.axis_index("s") * 2 + lax.axis_index("c")
    per = C // 32
    base = wid * per

    g = [None] * per
    s = [None] * per
    for i in range(min(D, per)):
        g[i] = pltpu.async_copy(labels3.at[base + i], buf.at[i % D], sems[i % D])
    for i in range(per):
        g[i].wait()
        s[i] = pltpu.async_copy(buf.at[i % D], lout.at[base + i], sems[D + i % D])
        if i + D < per:
            s[i].wait()
            g[i + D] = pltpu.async_copy(labels3.at[base + i + D],
                                        buf.at[i % D], sems[i % D])
    for i in range(max(0, per - D), per):
        s[i].wait()


def _win_body(B, S, J, T_L,
              xp, tp, lens, c, sos, labels3, xo, to,
              win, lens_s, c_s, gidx, idx64, idx16, wsem):
    del xp, tp
    L = 16
    wid = lax.axis_index("s") * 2 + lax.axis_index("c")
    b = wid % B
    kind = wid // B

    pltpu.sync_copy(lens, lens_s)
    pltpu.sync_copy(c, c_s)
    lane = lax.iota(jnp.int32, L)
    bvec = jnp.full((L,), 0, jnp.int32) + b
    lbv = plsc.load_gather(lens_s, [bvec])
    cb = jnp.max(jnp.where(lane == b, c_s[...], 0), axis=0)

    gidx[...] = bvec
    gl = pltpu.async_copy(labels3.at[cb], win.at[pl.ds(0, T_L)], wsem)
    gs = pltpu.async_copy(sos.at[gidx], win.at[pl.ds(T_L, L)], wsem)
    gl.wait()
    gs.wait()

    @pl.when(kind == 0)
    def _():
        base = b * (S + 1)
        for k in range(T_L // L):
            idx64[pl.ds(k * L, L)] = lbv + (base + 2 + k * L) + lane
        idx16[...] = jnp.where(lane == 0, lbv + base + 1,
                               jnp.full((L,), 0, jnp.int32) + base)
        s1 = pltpu.async_copy(win.at[pl.ds(0, T_L)], xo.at[idx64], wsem)
        s2 = pltpu.async_copy(win.at[pl.ds(T_L, L)], xo.at[idx16], wsem)
        s1.wait()
        s2.wait()

    @pl.when(kind == 1)
    def _():
        base = b * S
        for k in range(T_L // L):
            idx64[pl.ds(k * L, L)] = lbv + (base + 1 + k * L) + lane
        idx16[...] = jnp.where(lane == 0, lbv + base, lbv + base + T_L + 1)
        s1 = pltpu.async_copy(win.at[pl.ds(0, T_L)], to.at[idx64], wsem)
        s2 = pltpu.async_copy(win.at[pl.ds(T_L, L)], to.at[idx16], wsem)
        s1.wait()
        s2.wait()


def kernel(x, tgt, lens, c, sos, labels):
    B, S, J = x.shape
    C, T_L = labels.shape[0], labels.shape[1]

    lab_out = _plmpmd._mpmd_map(
        [(plsc.VectorSubcoreMesh(core_axis_name="c", subcore_axis_name="s"),
          lambda *refs: _lcopy_body(C, T_L, J, *refs))],
        jax.ShapeDtypeStruct((C, T_L, J), labels.dtype),
        scratch_types=[pltpu.VMEM((3, T_L, J), labels.dtype)]
        + [pltpu.SemaphoreType.DMA] * 6,
        compiler_params=pltpu.CompilerParams(needs_layout_passes=False),
    )(labels)

    x_pre, t_pre = pl.pallas_call(
        _bulk_body,
        grid=(B,),
        out_shape=(
            jax.ShapeDtypeStruct((B, S + 1, J), x.dtype),
            jax.ShapeDtypeStruct((B, S, J), tgt.dtype),
        ),
        in_specs=[
            pl.BlockSpec((1, S, J), lambda b: (b, 0, 0)),
            pl.BlockSpec((1, S, J), lambda b: (b, 0, 0)),
            pl.BlockSpec((1, 1, J), lambda b: (b, 0, 0)),
        ],
        out_specs=(
            pl.BlockSpec((1, S + 1, J), lambda b: (b, 0, 0)),
            pl.BlockSpec((1, S, J), lambda b: (b, 0, 0)),
        ),
    )(x, tgt, sos.reshape(B, 1, J))

    win_call = _plmpmd._mpmd_map(
        [(plsc.VectorSubcoreMesh(core_axis_name="c", subcore_axis_name="s"),
          lambda *refs: _win_body(B, S, J, T_L, *refs))],
        (
            jax.ShapeDtypeStruct((B * (S + 1), J), x.dtype),
            jax.ShapeDtypeStruct((B * S, J), tgt.dtype),
        ),
        input_output_aliases={0: 0, 1: 1},
        scratch_types=[
            pltpu.VMEM((T_L + 16, J), x.dtype),
            pltpu.VMEM((B,), jnp.int32),
            pltpu.VMEM((B,), jnp.int32),
            pltpu.VMEM((16,), jnp.int32),
            pltpu.VMEM((T_L,), jnp.int32),
            pltpu.VMEM((16,), jnp.int32),
            pltpu.SemaphoreType.DMA,
        ],
        compiler_params=pltpu.CompilerParams(needs_layout_passes=False),
    )
    x_out, tgt_out = win_call(
        x_pre.reshape(B * (S + 1), J), t_pre.reshape(B * S, J),
        lens, c, sos, labels)
    return (x_out.reshape(B, S + 1, J), tgt_out.reshape(B, S, J), lab_out)
